# Initial kernel scaffold; baseline (speedup 1.0000x reference)
import jax
import jax.numpy as jnp
from jax.experimental import pallas as pl


def kernel(x, gate_w, ew1, ew3, ew2, sw1, sw3, sw2):
    raise NotImplementedError("write your pallas kernel here")


# speedup vs baseline: 1.5508x; 1.5508x over previous
import functools

import jax
import jax.numpy as jnp
from jax import lax
from jax.experimental import pallas as pl
from jax.experimental.pallas import tpu as pltpu
from jax.experimental.pallas import tpu_sc as plsc

T = 2048
D = 1024
H = 2816
E = 8
BLK = 128
NBLK_R = T // BLK + E
PT = NBLK_R * BLK
NBLK_S = T // BLK


def _router_body(x_ref, gw_ref, slot_ref, topw_ref, be_ref):
    x = x_ref[...]
    gw = gw_ref[...]
    logits = jnp.dot(x, gw, preferred_element_type=jnp.float32)
    m = jnp.max(logits, axis=1, keepdims=True)
    s = jnp.sum(jnp.exp(logits - m), axis=1, keepdims=True)
    topw_ref[...] = 1.0 / s
    ei = lax.broadcasted_iota(jnp.int32, (T, E), 1)
    top_i = jnp.min(jnp.where(logits == m, ei, E), axis=1, keepdims=True)
    onehot = (ei == top_i).astype(jnp.float32)
    r_i = lax.broadcasted_iota(jnp.int32, (T, T), 0)
    c_i = lax.broadcasted_iota(jnp.int32, (T, T), 1)
    ltri = (c_i < r_i).astype(jnp.float32)
    ranks = jnp.dot(ltri, onehot, preferred_element_type=jnp.float32)
    rank_t = jnp.sum(ranks * onehot, axis=1, keepdims=True)
    counts = jnp.sum(onehot, axis=0, keepdims=True)
    padded = jnp.ceil(counts / BLK) * BLK
    eu = (lax.broadcasted_iota(jnp.int32, (E, E), 0)
          < lax.broadcasted_iota(jnp.int32, (E, E), 1)).astype(jnp.float32)
    off = jnp.dot(padded, eu, preferred_element_type=jnp.float32)
    off_t = jnp.sum(onehot * off, axis=1, keepdims=True)
    slot_ref[...] = (off_t + rank_t).astype(jnp.int32)
    ends = off + padded
    bstart = (lax.broadcasted_iota(jnp.int32, (NBLK_R, E), 0) * BLK
              ).astype(jnp.float32)
    be = jnp.sum((bstart >= ends).astype(jnp.int32), axis=1, keepdims=True)
    be_ref[...] = jnp.minimum(be, E - 1)


def _router(x, gate_w):
    return pl.pallas_call(
        _router_body,
        out_shape=(
            jax.ShapeDtypeStruct((T, 1), jnp.int32),
            jax.ShapeDtypeStruct((T, 1), jnp.float32),
            jax.ShapeDtypeStruct((NBLK_R, 1), jnp.int32),
        ),
    )(x, gate_w)


KC = 2
HC = H // KC


def _gffn_body(be_ref, xs_ref, w1_ref, w3_ref, w2_ref, tw_ref, out_ref):
    k = pl.program_id(1)
    xb = xs_ref[...]
    h1 = lax.dot_general(xb, w1_ref[0], (((1,), (1,)), ((), ())),
                         preferred_element_type=jnp.float32)
    h3 = lax.dot_general(xb, w3_ref[0], (((1,), (1,)), ((), ())),
                         preferred_element_type=jnp.float32)
    h = h1 * lax.logistic(h1) * h3
    ob = lax.dot_general(h, w2_ref[0], (((1,), (1,)), ((), ())),
                         preferred_element_type=jnp.float32)
    ob = ob * tw_ref[...]

    @pl.when(k == 0)
    def _init():
        out_ref[...] = ob

    @pl.when(k != 0)
    def _acc():
        out_ref[...] += ob


def _serp(i, k):
    return jnp.where(i % 2 == 0, k, KC - 1 - k)


def _grouped_ffn(be, xs, ew1, ew3, ew2, tws):
    grid_spec = pltpu.PrefetchScalarGridSpec(
        num_scalar_prefetch=1,
        grid=(NBLK_R, KC),
        in_specs=[
            pl.BlockSpec((BLK, D), lambda i, k, be_r: (i, 0)),
            pl.BlockSpec((1, HC, D),
                         lambda i, k, be_r: (be_r[i], _serp(i, k), 0)),
            pl.BlockSpec((1, HC, D),
                         lambda i, k, be_r: (be_r[i], _serp(i, k), 0)),
            pl.BlockSpec((1, D, HC),
                         lambda i, k, be_r: (be_r[i], 0, _serp(i, k))),
            pl.BlockSpec((BLK, 1), lambda i, k, be_r: (i, 0)),
        ],
        out_specs=pl.BlockSpec((BLK, D), lambda i, k, be_r: (i, 0)),
    )
    return pl.pallas_call(
        _gffn_body,
        grid_spec=grid_spec,
        out_shape=jax.ShapeDtypeStruct((PT, D), jnp.float32),
    )(be, xs, ew1, ew3, ew2, tws)


def _sffn_body(x_ref, w1_ref, w3_ref, w2_ref, yg_ref, out_ref):
    xb = x_ref[...]
    h1 = lax.dot_general(xb, w1_ref[...], (((1,), (1,)), ((), ())),
                         preferred_element_type=jnp.float32)
    h3 = lax.dot_general(xb, w3_ref[...], (((1,), (1,)), ((), ())),
                         preferred_element_type=jnp.float32)
    h = h1 * lax.logistic(h1) * h3
    ob = lax.dot_general(h, w2_ref[...], (((1,), (1,)), ((), ())),
                         preferred_element_type=jnp.float32)
    out_ref[...] = ob + yg_ref[...]


def _shared_ffn(x, sw1, sw3, sw2, yg):
    return pl.pallas_call(
        _sffn_body,
        grid=(NBLK_S,),
        in_specs=[
            pl.BlockSpec((BLK, D), lambda i: (i, 0)),
            pl.BlockSpec((H, D), lambda i: (0, 0)),
            pl.BlockSpec((H, D), lambda i: (0, 0)),
            pl.BlockSpec((D, H), lambda i: (0, 0)),
            pl.BlockSpec((BLK, D), lambda i: (i, 0)),
        ],
        out_specs=pl.BlockSpec((BLK, D), lambda i: (i, 0)),
        out_shape=jax.ShapeDtypeStruct((T, D), jnp.float32),
    )(x, sw1, sw3, sw2, yg)


def _dispatch(x, slot, topw):
    xs = jnp.zeros((PT, D), jnp.float32).at[slot].set(x)
    tws = jnp.zeros((PT,), jnp.float32).at[slot].set(topw)
    return xs, tws


def _gather_back(ys, slot):
    return ys[slot]


def kernel(x, gate_w, ew1, ew3, ew2, sw1, sw3, sw2):
    slot, topw, be = _router(x, gate_w)
    slot_f = slot[:, 0]
    xs, tws = _dispatch(x, slot_f, topw[:, 0])
    ys = _grouped_ffn(be[:, 0], xs, ew1, ew3, ew2, tws[:, None])
    yg = _gather_back(ys, slot_f)
    return _shared_ffn(x, sw1, sw3, sw2, yg)
